# Initial kernel scaffold; baseline (speedup 1.0000x reference)
import jax
import jax.numpy as jnp
from jax.experimental import pallas as pl


def kernel(dR, Z, neighbors, emb, fw1, fb1, fw2, fb2, in2f_w, f2out_w, f2out_b, dense_w, dense_b, aw1, ab1, aw2, ab2):
    raise NotImplementedError("write your pallas kernel here")


# speedup vs baseline: 1.5739x; 1.5739x over previous
import functools

import jax
import jax.numpy as jnp
import numpy as np
from jax import lax
from jax.experimental import pallas as pl
from jax.experimental.pallas import tpu as pltpu
from jax.experimental.pallas import tpu_sc as plsc

N_ATOMS = 10000
N_NBH = 32
N_ATOM_BASIS = 128
N_GAUSSIANS = 25
R_CUTOFF = 5.0
NPAD = 10240
BLK = 512
NW = 32
CW = 128
CH = NPAD // CW

_OFF = np.linspace(0.0, R_CUTOFF, N_GAUSSIANS).astype(np.float32)
_COEFF = np.float32(-0.5 / (_OFF[1] - _OFF[0]) ** 2)
_LOG2 = np.float32(np.log(2.0))


def _ssp(v):
    return jnp.maximum(v, 0.0) + jnp.log(1.0 + jnp.exp(-jnp.abs(v))) - _LOG2


def _sc_gather(y, idx3):
    mesh = plsc.VectorSubcoreMesh(
        core_axis_name="c", subcore_axis_name="s", num_cores=2, num_subcores=16
    )

    @functools.partial(
        pl.kernel,
        out_type=jax.ShapeDtypeStruct((NW, NPAD, 128), jnp.float32),
        mesh=mesh,
        scratch_types=[
            pltpu.VMEM((CH, CW), jnp.int32),
            pltpu.VMEM((CW, 128), jnp.float32),
            pltpu.VMEM((CW, 128), jnp.float32),
            pltpu.SemaphoreType.DMA,
            pltpu.SemaphoreType.DMA,
        ],
    )
    def gk(y_hbm, idx_hbm, out_hbm, idxv, buf0, buf1, sem0, sem1):
        w = lax.axis_index("s") * 2 + lax.axis_index("c")
        pltpu.sync_copy(idx_hbm.at[w], idxv)
        pltpu.async_copy(y_hbm.at[idxv.at[0]], buf0, sem0)

        def body(g, carry):
            j0 = g * 2
            pltpu.async_copy(y_hbm.at[idxv.at[j0 + 1]], buf1, sem1)
            pltpu.make_async_copy(y_hbm.at[idxv.at[j0]], buf0, sem0).wait()
            pltpu.sync_copy(buf0, out_hbm.at[w, pl.ds(j0 * CW, CW)])

            @pl.when(j0 + 2 < CH)
            def _():
                pltpu.async_copy(y_hbm.at[idxv.at[j0 + 2]], buf0, sem0)

            pltpu.make_async_copy(y_hbm.at[idxv.at[j0 + 1]], buf1, sem1).wait()
            pltpu.sync_copy(buf1, out_hbm.at[w, pl.ds((j0 + 1) * CW, CW)])
            return carry

        lax.fori_loop(0, CH // 2, body, 0)

    return gk(y, idx3)


def _k0_body(z_ref, emb_ref, in2f_ref, x_ref, y_ref):
    z = z_ref[...]
    ids = lax.broadcasted_iota(jnp.int32, (1, emb_ref.shape[0]), 1)
    oh = (z == ids).astype(jnp.float32)
    x = jnp.dot(oh, emb_ref[...], preferred_element_type=jnp.float32)
    x_ref[...] = x
    y_ref[...] = jnp.dot(x, in2f_ref[...], preferred_element_type=jnp.float32)


def _cfconv(dr, yg_ref, fw1, fb1, fw2, fb2):
    cut = 0.5 * (jnp.cos(dr * (np.pi / R_CUTOFF)) + 1.0)
    cut = cut * (dr < R_CUTOFF).astype(jnp.float32)
    off = lax.broadcasted_iota(jnp.int32, (1, N_GAUSSIANS), 1).astype(
        jnp.float32) * np.float32(_OFF[1] - _OFF[0])
    acc = jnp.zeros((dr.shape[0], 128), jnp.float32)
    for k in range(N_NBH):
        drk = dr[:, k : k + 1]
        f = jnp.exp(_COEFF * (drk - off) ** 2)
        h1 = _ssp(jnp.dot(f, fw1, preferred_element_type=jnp.float32) + fb1)
        wk = jnp.dot(h1, fw2, preferred_element_type=jnp.float32) + fb2
        acc = acc + wk * yg_ref[k] * cut[:, k : k + 1]
    return acc


def _mid_body(dr_ref, yg_ref, x_ref, fw1_ref, fb1_ref, fw2_ref, fb2_ref,
              f2w_ref, f2b_ref, dw_ref, db_ref, in2f_ref, xo_ref, yo_ref):
    agg = _cfconv(dr_ref[...], yg_ref, fw1_ref[...], fb1_ref[...],
                  fw2_ref[...], fb2_ref[...])
    h = _ssp(jnp.dot(agg, f2w_ref[...], preferred_element_type=jnp.float32) + f2b_ref[...])
    v = jnp.dot(h, dw_ref[...], preferred_element_type=jnp.float32) + db_ref[...]
    xn = x_ref[...] + v
    xo_ref[...] = xn
    yo_ref[...] = jnp.dot(xn, in2f_ref[...], preferred_element_type=jnp.float32)


def _last_body(dr_ref, yg_ref, x_ref, fw1_ref, fb1_ref, fw2_ref, fb2_ref,
               f2w_ref, f2b_ref, dw_ref, db_ref, aw1_ref, ab1_ref, aw2_ref,
               ab2_ref, e_ref):
    agg = _cfconv(dr_ref[...], yg_ref, fw1_ref[...], fb1_ref[...],
                  fw2_ref[...], fb2_ref[...])
    h = _ssp(jnp.dot(agg, f2w_ref[...], preferred_element_type=jnp.float32) + f2b_ref[...])
    v = jnp.dot(h, dw_ref[...], preferred_element_type=jnp.float32) + db_ref[...]
    xn = x_ref[...] + v
    t = _ssp(jnp.dot(xn, aw1_ref[...], preferred_element_type=jnp.float32) + ab1_ref[...])
    yi = jnp.dot(t, aw2_ref[...], preferred_element_type=jnp.float32) + ab2_ref[...]
    i = pl.program_id(0)
    gid = i * BLK + lax.broadcasted_iota(jnp.int32, (BLK, 1), 0)
    yi = jnp.where(gid < N_ATOMS, yi, 0.0)

    @pl.when(i == 0)
    def _():
        e_ref[...] = jnp.zeros((1, 1), jnp.float32)

    e_ref[...] += jnp.sum(yi).reshape(1, 1)


def _full(shape):
    return pl.BlockSpec(shape, lambda i: (0,) * len(shape))


_ROW = pl.BlockSpec((BLK, 128), lambda i: (i, 0))
_SEQ = pltpu.CompilerParams(dimension_semantics=("arbitrary",))
_GRID = NPAD // BLK


def _tc_k0(zc, emb_p, in2f0):
    return pl.pallas_call(
        _k0_body,
        grid=(_GRID,),
        in_specs=[
            pl.BlockSpec((BLK, 1), lambda i: (i, 0)),
            _full(emb_p.shape),
            _full((128, 128)),
        ],
        out_specs=[_ROW, _ROW],
        out_shape=[
            jax.ShapeDtypeStruct((NPAD, 128), jnp.float32),
            jax.ShapeDtypeStruct((NPAD, 128), jnp.float32),
        ],
        compiler_params=_SEQ,
    )(zc, emb_p, in2f0)


def _tc_mid(dr, yg, x, fw1, fb1, fw2, fb2, f2w, f2b, dw, db, in2f_next):
    return pl.pallas_call(
        _mid_body,
        grid=(_GRID,),
        in_specs=[
            pl.BlockSpec((BLK, N_NBH), lambda i: (i, 0)),
            pl.BlockSpec((N_NBH, BLK, 128), lambda i: (0, i, 0)),
            _ROW,
            _full((N_GAUSSIANS, 128)), _full((1, 128)),
            _full((128, 128)), _full((1, 128)),
            _full((128, 128)), _full((1, 128)),
            _full((128, 128)), _full((1, 128)),
            _full((128, 128)),
        ],
        out_specs=[_ROW, _ROW],
        out_shape=[
            jax.ShapeDtypeStruct((NPAD, 128), jnp.float32),
            jax.ShapeDtypeStruct((NPAD, 128), jnp.float32),
        ],
        compiler_params=_SEQ,
    )(dr, yg, x, fw1, fb1, fw2, fb2, f2w, f2b, dw, db, in2f_next)


def _tc_last(dr, yg, x, fw1, fb1, fw2, fb2, f2w, f2b, dw, db, aw1, ab1, aw2, ab2):
    return pl.pallas_call(
        _last_body,
        grid=(_GRID,),
        in_specs=[
            pl.BlockSpec((BLK, N_NBH), lambda i: (i, 0)),
            pl.BlockSpec((N_NBH, BLK, 128), lambda i: (0, i, 0)),
            _ROW,
            _full((N_GAUSSIANS, 128)), _full((1, 128)),
            _full((128, 128)), _full((1, 128)),
            _full((128, 128)), _full((1, 128)),
            _full((128, 128)), _full((1, 128)),
            _full((128, 64)), _full((1, 64)),
            _full((64, 1)), _full((1, 1)),
        ],
        out_specs=pl.BlockSpec((1, 1), lambda i: (0, 0)),
        out_shape=jax.ShapeDtypeStruct((1, 1), jnp.float32),
        compiler_params=_SEQ,
    )(dr, yg, x, fw1, fb1, fw2, fb2, f2w, f2b, dw, db, aw1, ab1, aw2, ab2)


def kernel(dR, Z, neighbors, emb, fw1, fb1, fw2, fb2, in2f_w, f2out_w,
           f2out_b, dense_w, dense_b, aw1, ab1, aw2, ab2):
    pad = NPAD - N_ATOMS
    dr_p = jnp.pad(dR, ((0, pad), (0, 0)))
    z_p = jnp.pad(Z, (0, pad)).reshape(NPAD, 1)
    nb_p = jnp.pad(neighbors, ((0, pad), (0, 0)))
    idx3 = nb_p.T.reshape(NW, CH, CW)
    mz = emb.shape[0]
    mzp = (-mz) % 8
    emb_p = jnp.pad(emb, ((0, mzp), (0, 0)))
    r2 = lambda b: b.reshape(1, -1)

    x, y = _tc_k0(z_p, emb_p, in2f_w[0])
    for i in range(3):
        yg = _sc_gather(y, idx3)
        if i < 2:
            x, y = _tc_mid(
                dr_p, yg, x, fw1[i], r2(fb1[i]), fw2[i], r2(fb2[i]),
                f2out_w[i], r2(f2out_b[i]), dense_w[i], r2(dense_b[i]),
                in2f_w[i + 1])
        else:
            e = _tc_last(
                dr_p, yg, x, fw1[i], r2(fb1[i]), fw2[i], r2(fb2[i]),
                f2out_w[i], r2(f2out_b[i]), dense_w[i], r2(dense_b[i]),
                aw1, r2(ab1), aw2, ab2.reshape(1, 1))
    return e[0, 0]


# speedup vs baseline: 1.5799x; 1.0038x over previous
import functools

import jax
import jax.numpy as jnp
import numpy as np
from jax import lax
from jax.experimental import pallas as pl
from jax.experimental.pallas import tpu as pltpu
from jax.experimental.pallas import tpu_sc as plsc

N_ATOMS = 10000
N_NBH = 32
N_ATOM_BASIS = 128
N_GAUSSIANS = 25
R_CUTOFF = 5.0
NPAD = 10240
BLK = 512
NW = 32
CW = 128
CH = NPAD // CW

_OFF = np.linspace(0.0, R_CUTOFF, N_GAUSSIANS).astype(np.float32)
_COEFF = np.float32(-0.5 / (_OFF[1] - _OFF[0]) ** 2)
_LOG2 = np.float32(np.log(2.0))


def _ssp(v):
    return jnp.maximum(v, 0.0) + jnp.log(1.0 + jnp.exp(-jnp.abs(v))) - _LOG2


def _sc_gather(y, idx3):
    mesh = plsc.VectorSubcoreMesh(
        core_axis_name="c", subcore_axis_name="s", num_cores=2, num_subcores=16
    )

    nbuf = 4

    @functools.partial(
        pl.kernel,
        out_type=jax.ShapeDtypeStruct((NW, NPAD, 128), jnp.float32),
        mesh=mesh,
        scratch_types=[
            pltpu.VMEM((CH, CW), jnp.int32),
        ] + [pltpu.VMEM((CW, 128), jnp.float32) for _ in range(nbuf)]
          + [pltpu.SemaphoreType.DMA for _ in range(2 * nbuf)],
    )
    def gk(y_hbm, idx_hbm, out_hbm, idxv, *scr):
        bufs = scr[:nbuf]
        gs = scr[nbuf : 2 * nbuf]
        ws = scr[2 * nbuf :]
        w = lax.axis_index("s") * 2 + lax.axis_index("c")
        pltpu.sync_copy(idx_hbm.at[w], idxv)
        pltpu.async_copy(y_hbm.at[idxv.at[0]], bufs[0], gs[0])
        pltpu.async_copy(y_hbm.at[idxv.at[1]], bufs[1], gs[1])

        def body(t, carry):
            for u in range(nbuf):
                j = t * nbuf + u
                u2 = (u + 2) % nbuf

                @pl.when(j >= 2)
                def _():
                    pltpu.make_async_copy(
                        bufs[u2], out_hbm.at[w, pl.ds(0, CW)], ws[u2]
                    ).wait()

                @pl.when(j + 2 < CH)
                def _():
                    pltpu.async_copy(y_hbm.at[idxv.at[j + 2]], bufs[u2], gs[u2])

                pltpu.make_async_copy(
                    y_hbm.at[idxv.at[j]], bufs[u], gs[u]
                ).wait()
                pltpu.async_copy(
                    bufs[u], out_hbm.at[w, pl.ds(j * CW, CW)], ws[u]
                )
            return carry

        lax.fori_loop(0, CH // nbuf, body, 0)
        for u in (2, 3):
            pltpu.make_async_copy(
                bufs[u], out_hbm.at[w, pl.ds(0, CW)], ws[u]
            ).wait()

    return gk(y, idx3)


def _k0_body(z_ref, emb_ref, in2f_ref, x_ref, y_ref):
    z = z_ref[...]
    ids = lax.broadcasted_iota(jnp.int32, (1, emb_ref.shape[0]), 1)
    oh = (z == ids).astype(jnp.float32)
    x = jnp.dot(oh, emb_ref[...], preferred_element_type=jnp.float32)
    x_ref[...] = x
    y_ref[...] = jnp.dot(x, in2f_ref[...], preferred_element_type=jnp.float32)


def _cfconv(dr, yg_ref, fw1, fb1, fw2, fb2):
    cut = 0.5 * (jnp.cos(dr * (np.pi / R_CUTOFF)) + 1.0)
    cut = cut * (dr < R_CUTOFF).astype(jnp.float32)
    off = lax.broadcasted_iota(jnp.int32, (1, N_GAUSSIANS), 1).astype(
        jnp.float32) * np.float32(_OFF[1] - _OFF[0])
    acc = jnp.zeros((dr.shape[0], 128), jnp.float32)
    for k in range(N_NBH):
        drk = dr[:, k : k + 1]
        f = jnp.exp(_COEFF * (drk - off) ** 2)
        h1 = _ssp(jnp.dot(f, fw1, preferred_element_type=jnp.float32) + fb1)
        wk = jnp.dot(h1, fw2, preferred_element_type=jnp.float32) + fb2
        acc = acc + wk * yg_ref[k] * cut[:, k : k + 1]
    return acc


def _mid_body(dr_ref, yg_ref, x_ref, fw1_ref, fb1_ref, fw2_ref, fb2_ref,
              f2w_ref, f2b_ref, dw_ref, db_ref, in2f_ref, xo_ref, yo_ref):
    agg = _cfconv(dr_ref[...], yg_ref, fw1_ref[...], fb1_ref[...],
                  fw2_ref[...], fb2_ref[...])
    h = _ssp(jnp.dot(agg, f2w_ref[...], preferred_element_type=jnp.float32) + f2b_ref[...])
    v = jnp.dot(h, dw_ref[...], preferred_element_type=jnp.float32) + db_ref[...]
    xn = x_ref[...] + v
    xo_ref[...] = xn
    yo_ref[...] = jnp.dot(xn, in2f_ref[...], preferred_element_type=jnp.float32)


def _last_body(dr_ref, yg_ref, x_ref, fw1_ref, fb1_ref, fw2_ref, fb2_ref,
               f2w_ref, f2b_ref, dw_ref, db_ref, aw1_ref, ab1_ref, aw2_ref,
               ab2_ref, e_ref):
    agg = _cfconv(dr_ref[...], yg_ref, fw1_ref[...], fb1_ref[...],
                  fw2_ref[...], fb2_ref[...])
    h = _ssp(jnp.dot(agg, f2w_ref[...], preferred_element_type=jnp.float32) + f2b_ref[...])
    v = jnp.dot(h, dw_ref[...], preferred_element_type=jnp.float32) + db_ref[...]
    xn = x_ref[...] + v
    t = _ssp(jnp.dot(xn, aw1_ref[...], preferred_element_type=jnp.float32) + ab1_ref[...])
    yi = jnp.dot(t, aw2_ref[...], preferred_element_type=jnp.float32) + ab2_ref[...]
    i = pl.program_id(0)
    gid = i * BLK + lax.broadcasted_iota(jnp.int32, (BLK, 1), 0)
    yi = jnp.where(gid < N_ATOMS, yi, 0.0)

    @pl.when(i == 0)
    def _():
        e_ref[...] = jnp.zeros((1, 1), jnp.float32)

    e_ref[...] += jnp.sum(yi).reshape(1, 1)


def _full(shape):
    return pl.BlockSpec(shape, lambda i: (0,) * len(shape))


_ROW = pl.BlockSpec((BLK, 128), lambda i: (i, 0))
_SEQ = pltpu.CompilerParams(dimension_semantics=("arbitrary",))
_GRID = NPAD // BLK


def _tc_k0(zc, emb_p, in2f0):
    return pl.pallas_call(
        _k0_body,
        grid=(_GRID,),
        in_specs=[
            pl.BlockSpec((BLK, 1), lambda i: (i, 0)),
            _full(emb_p.shape),
            _full((128, 128)),
        ],
        out_specs=[_ROW, _ROW],
        out_shape=[
            jax.ShapeDtypeStruct((NPAD, 128), jnp.float32),
            jax.ShapeDtypeStruct((NPAD, 128), jnp.float32),
        ],
        compiler_params=_SEQ,
    )(zc, emb_p, in2f0)


def _tc_mid(dr, yg, x, fw1, fb1, fw2, fb2, f2w, f2b, dw, db, in2f_next):
    return pl.pallas_call(
        _mid_body,
        grid=(_GRID,),
        in_specs=[
            pl.BlockSpec((BLK, N_NBH), lambda i: (i, 0)),
            pl.BlockSpec((N_NBH, BLK, 128), lambda i: (0, i, 0)),
            _ROW,
            _full((N_GAUSSIANS, 128)), _full((1, 128)),
            _full((128, 128)), _full((1, 128)),
            _full((128, 128)), _full((1, 128)),
            _full((128, 128)), _full((1, 128)),
            _full((128, 128)),
        ],
        out_specs=[_ROW, _ROW],
        out_shape=[
            jax.ShapeDtypeStruct((NPAD, 128), jnp.float32),
            jax.ShapeDtypeStruct((NPAD, 128), jnp.float32),
        ],
        compiler_params=_SEQ,
    )(dr, yg, x, fw1, fb1, fw2, fb2, f2w, f2b, dw, db, in2f_next)


def _tc_last(dr, yg, x, fw1, fb1, fw2, fb2, f2w, f2b, dw, db, aw1, ab1, aw2, ab2):
    return pl.pallas_call(
        _last_body,
        grid=(_GRID,),
        in_specs=[
            pl.BlockSpec((BLK, N_NBH), lambda i: (i, 0)),
            pl.BlockSpec((N_NBH, BLK, 128), lambda i: (0, i, 0)),
            _ROW,
            _full((N_GAUSSIANS, 128)), _full((1, 128)),
            _full((128, 128)), _full((1, 128)),
            _full((128, 128)), _full((1, 128)),
            _full((128, 128)), _full((1, 128)),
            _full((128, 64)), _full((1, 64)),
            _full((64, 1)), _full((1, 1)),
        ],
        out_specs=pl.BlockSpec((1, 1), lambda i: (0, 0)),
        out_shape=jax.ShapeDtypeStruct((1, 1), jnp.float32),
        compiler_params=_SEQ,
    )(dr, yg, x, fw1, fb1, fw2, fb2, f2w, f2b, dw, db, aw1, ab1, aw2, ab2)


def kernel(dR, Z, neighbors, emb, fw1, fb1, fw2, fb2, in2f_w, f2out_w,
           f2out_b, dense_w, dense_b, aw1, ab1, aw2, ab2):
    pad = NPAD - N_ATOMS
    dr_p = jnp.pad(dR, ((0, pad), (0, 0)))
    z_p = jnp.pad(Z, (0, pad)).reshape(NPAD, 1)
    nb_p = jnp.pad(neighbors, ((0, pad), (0, 0)))
    idx3 = nb_p.T.reshape(NW, CH, CW)
    mz = emb.shape[0]
    mzp = (-mz) % 8
    emb_p = jnp.pad(emb, ((0, mzp), (0, 0)))
    r2 = lambda b: b.reshape(1, -1)

    x, y = _tc_k0(z_p, emb_p, in2f_w[0])
    for i in range(3):
        yg = _sc_gather(y, idx3)
        if i < 2:
            x, y = _tc_mid(
                dr_p, yg, x, fw1[i], r2(fb1[i]), fw2[i], r2(fb2[i]),
                f2out_w[i], r2(f2out_b[i]), dense_w[i], r2(dense_b[i]),
                in2f_w[i + 1])
        else:
            e = _tc_last(
                dr_p, yg, x, fw1[i], r2(fb1[i]), fw2[i], r2(fb2[i]),
                f2out_w[i], r2(f2out_b[i]), dense_w[i], r2(dense_b[i]),
                aw1, r2(ab1), aw2, ab2.reshape(1, 1))
    return e[0, 0]
